# Initial kernel scaffold; baseline (speedup 1.0000x reference)
import jax
import jax.numpy as jnp
from jax.experimental import pallas as pl


def kernel(x, im_info):
    raise NotImplementedError("write your pallas kernel here")


# speedup vs baseline: 23.2619x; 23.2619x over previous
import jax
import jax.numpy as jnp
from jax.experimental import pallas as pl
from jax.experimental.pallas import tpu as pltpu

_A, _C, _H, _W = 5, 20, 64, 64
_HW = _H * _W
_N = _A * _HW
_ROWS = _N // 128
_SROWS = _HW // 128
_NMS_THR = 0.45
_PRE_THR = 0.005
_BIASES = ((1.08, 1.19), (3.42, 4.41), (6.63, 11.38), (9.42, 5.11), (16.62, 10.52))


def _yolo_body(x_ref, im_ref, box_ref, tp_ref, objf_ref):
    im_h = im_ref[0, 0]
    im_w = im_ref[0, 1]
    r_i = jax.lax.broadcasted_iota(jnp.int32, (_SROWS, 128), 0)
    c_i = jax.lax.broadcasted_iota(jnp.int32, (_SROWS, 128), 1)
    sidx = r_i * 128 + c_i
    gxf = (sidx % _W).astype(jnp.float32)
    gyf = (sidx // _W).astype(jnp.float32)

    x1s, y1s, x2s, y2s, ss = [], [], [], [], []
    for a in range(_A):
        tx = x_ref[2 * a]
        ty = x_ref[2 * a + 1]
        tw = x_ref[2 * _A + 2 * a]
        th = x_ref[2 * _A + 2 * a + 1]
        to = x_ref[4 * _A + a]
        obj = jax.nn.sigmoid(to)
        cx = (jax.nn.sigmoid(tx) + gxf) / _W
        cy = (jax.nn.sigmoid(ty) + gyf) / _H
        bw = jnp.exp(tw) * jnp.float32(_BIASES[a][0]) / _W
        bh = jnp.exp(th) * jnp.float32(_BIASES[a][1]) / _H
        x1s.append(jnp.clip((cx - bw * 0.5) * im_w, 0.0, im_w - 1.0))
        y1s.append(jnp.clip((cy - bh * 0.5) * im_h, 0.0, im_h - 1.0))
        x2s.append(jnp.clip((cx + bw * 0.5) * im_w, 0.0, im_w - 1.0))
        y2s.append(jnp.clip((cy + bh * 0.5) * im_h, 0.0, im_h - 1.0))
        ss.append(jnp.where(obj > _PRE_THR, obj, 0.0))
        conf = x_ref[5 * _A + _C * a: 5 * _A + _C * (a + 1)]
        mx = jnp.max(conf, axis=0, keepdims=True)
        e = jnp.exp(conf - mx)
        cp = e / jnp.sum(e, axis=0, keepdims=True)
        tp_ref[a] = cp * obj[None]

    X1 = jnp.concatenate(x1s, 0)
    Y1 = jnp.concatenate(y1s, 0)
    X2 = jnp.concatenate(x2s, 0)
    Y2 = jnp.concatenate(y2s, 0)
    S = jnp.concatenate(ss, 0)
    box_ref[0] = X1
    box_ref[1] = Y1
    box_ref[2] = X2
    box_ref[3] = Y2

    AREA = jnp.clip(X2 - X1, 0.0, None) * jnp.clip(Y2 - Y1, 0.0, None)
    IDX = (jax.lax.broadcasted_iota(jnp.int32, (_ROWS, 128), 0) * 128
           + jax.lax.broadcasted_iota(jnp.int32, (_ROWS, 128), 1))

    def pick(masked):
        m = jnp.max(masked)
        k = jnp.min(jnp.where(masked == m, IDX, _N))
        return m, k

    masked0 = jnp.where(S > 0.0, S, -1.0)
    m0, k0 = pick(masked0)

    def cond(c):
        return c[1] > 0.0

    def body(c):
        masked, _, k, keep = c
        sel = IDX == k
        gx1 = jnp.max(jnp.where(sel, X1, -1.0))
        gy1 = jnp.max(jnp.where(sel, Y1, -1.0))
        gx2 = jnp.max(jnp.where(sel, X2, -1.0))
        gy2 = jnp.max(jnp.where(sel, Y2, -1.0))
        ga = jnp.max(jnp.where(sel, AREA, -1.0))
        xx1 = jnp.maximum(gx1, X1)
        yy1 = jnp.maximum(gy1, Y1)
        xx2 = jnp.minimum(gx2, X2)
        yy2 = jnp.minimum(gy2, Y2)
        inter = jnp.clip(xx2 - xx1, 0.0, None) * jnp.clip(yy2 - yy1, 0.0, None)
        iou = inter / (ga + AREA - inter + 1e-9)
        keep = jnp.where(sel, 1.0, keep)
        masked = jnp.where(sel | (iou > _NMS_THR), -1.0, masked)
        m2, k2 = pick(masked)
        return masked, m2, k2, keep

    keep0 = jnp.zeros((_ROWS, 128), jnp.float32)
    _, _, _, keep = jax.lax.while_loop(cond, body, (masked0, m0, k0, keep0))
    objf_ref[:] = S * keep


def _run(x2d, im_info, interpret=False):
    return pl.pallas_call(
        _yolo_body,
        out_shape=[
            jax.ShapeDtypeStruct((4, _ROWS, 128), jnp.float32),
            jax.ShapeDtypeStruct((_A, _C, _SROWS, 128), jnp.float32),
            jax.ShapeDtypeStruct((_ROWS, 128), jnp.float32),
        ],
        in_specs=[
            pl.BlockSpec(memory_space=pltpu.VMEM),
            pl.BlockSpec(memory_space=pltpu.SMEM),
        ],
        interpret=interpret,
    )(x2d, im_info)


def kernel(x, im_info):
    x2d = x.reshape(125, _SROWS, 128)
    box4, tp, objf = _run(x2d, im_info)
    flat_boxes = box4.reshape(4, _N).T[None]
    tpf = tp.reshape(_A, _C, _HW).transpose(0, 2, 1).reshape(1, _N, _C)
    prob = jnp.concatenate([tpf, objf.reshape(1, _N, 1)], axis=-1)
    return prob, flat_boxes


# speedup vs baseline: 25.3377x; 1.0892x over previous
import jax
import jax.numpy as jnp
from jax.experimental import pallas as pl
from jax.experimental.pallas import tpu as pltpu

_A, _C, _H, _W = 5, 20, 64, 64
_HW = _H * _W
_N = _A * _HW
_ROWS = _N // 128
_SROWS = _HW // 128
_NMS_THR = 0.45
_PRE_THR = 0.005
_BIASES = ((1.08, 1.19), (3.42, 4.41), (6.63, 11.38), (9.42, 5.11), (16.62, 10.52))


def _yolo_body(x_ref, im_ref, box_ref, tp_ref, objf_ref, scr):
    im_h = im_ref[0, 0]
    im_w = im_ref[0, 1]
    r_i = jax.lax.broadcasted_iota(jnp.int32, (_SROWS, 128), 0)
    c_i = jax.lax.broadcasted_iota(jnp.int32, (_SROWS, 128), 1)
    sidx = r_i * 128 + c_i
    gxf = (sidx % _W).astype(jnp.float32)
    gyf = (sidx // _W).astype(jnp.float32)

    x1s, y1s, x2s, y2s, ss = [], [], [], [], []
    for a in range(_A):
        tx = x_ref[2 * a]
        ty = x_ref[2 * a + 1]
        tw = x_ref[2 * _A + 2 * a]
        th = x_ref[2 * _A + 2 * a + 1]
        to = x_ref[4 * _A + a]
        obj = jax.nn.sigmoid(to)
        cx = (jax.nn.sigmoid(tx) + gxf) / _W
        cy = (jax.nn.sigmoid(ty) + gyf) / _H
        bw = jnp.exp(tw) * jnp.float32(_BIASES[a][0]) / _W
        bh = jnp.exp(th) * jnp.float32(_BIASES[a][1]) / _H
        x1s.append(jnp.clip((cx - bw * 0.5) * im_w, 0.0, im_w - 1.0))
        y1s.append(jnp.clip((cy - bh * 0.5) * im_h, 0.0, im_h - 1.0))
        x2s.append(jnp.clip((cx + bw * 0.5) * im_w, 0.0, im_w - 1.0))
        y2s.append(jnp.clip((cy + bh * 0.5) * im_h, 0.0, im_h - 1.0))
        ss.append(jnp.where(obj > _PRE_THR, obj, 0.0))
        conf = x_ref[5 * _A + _C * a: 5 * _A + _C * (a + 1)]
        mx = jnp.max(conf, axis=0, keepdims=True)
        e = jnp.exp(conf - mx)
        cp = e / jnp.sum(e, axis=0, keepdims=True)
        tp_ref[a] = cp * obj[None]

    X1 = jnp.concatenate(x1s, 0)
    Y1 = jnp.concatenate(y1s, 0)
    X2 = jnp.concatenate(x2s, 0)
    Y2 = jnp.concatenate(y2s, 0)
    S = jnp.concatenate(ss, 0)
    box_ref[0] = X1
    box_ref[1] = Y1
    box_ref[2] = X2
    box_ref[3] = Y2

    AREA = jnp.clip(X2 - X1, 0.0, None) * jnp.clip(Y2 - Y1, 0.0, None)
    IDX = (jax.lax.broadcasted_iota(jnp.int32, (_ROWS, 128), 0) * 128
           + jax.lax.broadcasted_iota(jnp.int32, (_ROWS, 128), 1))
    scr[0 * _ROWS:1 * _ROWS] = X1
    scr[1 * _ROWS:2 * _ROWS] = Y1
    scr[2 * _ROWS:3 * _ROWS] = X2
    scr[3 * _ROWS:4 * _ROWS] = Y2
    scr[4 * _ROWS:5 * _ROWS] = AREA

    def pick(masked):
        m = jnp.max(masked)
        k = jnp.min(jnp.where(masked == m, IDX, _N))
        return m, k

    masked0 = jnp.where(S > 0.0, S, -1.0)
    m0, k0 = pick(masked0)

    def cond(c):
        return c[1] > 0.0

    def body(c):
        masked, _, k = c
        r = jax.lax.shift_right_logical(k, 7)
        col = jax.lax.bitwise_and(k, 127)
        rows = jnp.concatenate(
            [scr[pl.ds(i * _ROWS + r, 1), :] for i in range(5)], axis=0)
        rolled = pltpu.roll(rows, (128 - col) & 127, 1)
        gx1 = rolled[0, 0]
        gy1 = rolled[1, 0]
        gx2 = rolled[2, 0]
        gy2 = rolled[3, 0]
        ga = rolled[4, 0]
        xx1 = jnp.maximum(gx1, X1)
        yy1 = jnp.maximum(gy1, Y1)
        xx2 = jnp.minimum(gx2, X2)
        yy2 = jnp.minimum(gy2, Y2)
        inter = jnp.clip(xx2 - xx1, 0.0, None) * jnp.clip(yy2 - yy1, 0.0, None)
        iou = inter / (ga + AREA - inter + 1e-9)
        masked = jnp.where(IDX == k, -2.0,
                           jnp.where(iou > _NMS_THR, -1.0, masked))
        m2, k2 = pick(masked)
        return masked, m2, k2

    masked, _, _ = jax.lax.while_loop(cond, body, (masked0, m0, k0))
    objf_ref[:] = jnp.where(masked == -2.0, S, 0.0)


def _run(x2d, im_info, interpret=False):
    return pl.pallas_call(
        _yolo_body,
        out_shape=[
            jax.ShapeDtypeStruct((4, _ROWS, 128), jnp.float32),
            jax.ShapeDtypeStruct((_A, _C, _SROWS, 128), jnp.float32),
            jax.ShapeDtypeStruct((_ROWS, 128), jnp.float32),
        ],
        in_specs=[
            pl.BlockSpec(memory_space=pltpu.VMEM),
            pl.BlockSpec(memory_space=pltpu.SMEM),
        ],
        scratch_shapes=[pltpu.VMEM((5 * _ROWS, 128), jnp.float32)],
        interpret=interpret,
    )(x2d, im_info)


def kernel(x, im_info):
    x2d = x.reshape(125, _SROWS, 128)
    box4, tp, objf = _run(x2d, im_info)
    flat_boxes = box4.reshape(4, _N).T[None]
    tpf = tp.reshape(_A, _C, _HW).transpose(0, 2, 1).reshape(1, _N, _C)
    prob = jnp.concatenate([tpf, objf.reshape(1, _N, 1)], axis=-1)
    return prob, flat_boxes


# speedup vs baseline: 26.5108x; 1.0463x over previous
import jax
import jax.numpy as jnp
from jax.experimental import pallas as pl
from jax.experimental.pallas import tpu as pltpu

_A, _C, _H, _W = 5, 20, 64, 64
_HW = _H * _W
_N = _A * _HW
_ROWS = _N // 128
_SROWS = _HW // 128
_NMS_THR = 0.45
_PRE_THR = 0.005
_BIASES = ((1.08, 1.19), (3.42, 4.41), (6.63, 11.38), (9.42, 5.11), (16.62, 10.52))


def _yolo_body(x_ref, im_ref, box_ref, tp_ref, objf_ref, scr):
    im_h = im_ref[0, 0]
    im_w = im_ref[0, 1]
    r_i = jax.lax.broadcasted_iota(jnp.int32, (_SROWS, 128), 0)
    c_i = jax.lax.broadcasted_iota(jnp.int32, (_SROWS, 128), 1)
    sidx = r_i * 128 + c_i
    gxf = (sidx % _W).astype(jnp.float32)
    gyf = (sidx // _W).astype(jnp.float32)

    x1s, y1s, x2s, y2s, ss = [], [], [], [], []
    for a in range(_A):
        tx = x_ref[2 * a]
        ty = x_ref[2 * a + 1]
        tw = x_ref[2 * _A + 2 * a]
        th = x_ref[2 * _A + 2 * a + 1]
        to = x_ref[4 * _A + a]
        obj = jax.nn.sigmoid(to)
        cx = (jax.nn.sigmoid(tx) + gxf) / _W
        cy = (jax.nn.sigmoid(ty) + gyf) / _H
        bw = jnp.exp(tw) * jnp.float32(_BIASES[a][0]) / _W
        bh = jnp.exp(th) * jnp.float32(_BIASES[a][1]) / _H
        x1s.append(jnp.clip((cx - bw * 0.5) * im_w, 0.0, im_w - 1.0))
        y1s.append(jnp.clip((cy - bh * 0.5) * im_h, 0.0, im_h - 1.0))
        x2s.append(jnp.clip((cx + bw * 0.5) * im_w, 0.0, im_w - 1.0))
        y2s.append(jnp.clip((cy + bh * 0.5) * im_h, 0.0, im_h - 1.0))
        ss.append(jnp.where(obj > _PRE_THR, obj, 0.0))
        conf = x_ref[5 * _A + _C * a: 5 * _A + _C * (a + 1)]
        mx = jnp.max(conf, axis=0, keepdims=True)
        e = jnp.exp(conf - mx)
        cp = e / jnp.sum(e, axis=0, keepdims=True)
        tp_ref[a] = cp * obj[None]

    X1 = jnp.concatenate(x1s, 0)
    Y1 = jnp.concatenate(y1s, 0)
    X2 = jnp.concatenate(x2s, 0)
    Y2 = jnp.concatenate(y2s, 0)
    S = jnp.concatenate(ss, 0)
    box_ref[0] = X1
    box_ref[1] = Y1
    box_ref[2] = X2
    box_ref[3] = Y2

    AREA = jnp.clip(X2 - X1, 0.0, None) * jnp.clip(Y2 - Y1, 0.0, None)
    IDX = (jax.lax.broadcasted_iota(jnp.int32, (_ROWS, 128), 0) * 128
           + jax.lax.broadcasted_iota(jnp.int32, (_ROWS, 128), 1))
    scr[0 * _ROWS:1 * _ROWS] = X1
    scr[1 * _ROWS:2 * _ROWS] = Y1
    scr[2 * _ROWS:3 * _ROWS] = X2
    scr[3 * _ROWS:4 * _ROWS] = Y2
    scr[4 * _ROWS:5 * _ROWS] = AREA

    def pick(masked):
        m = jnp.max(masked)
        k = jnp.min(jnp.where(masked == m, IDX, _N))
        return m, k

    def pick2(masked):
        m1, k1 = pick(masked)
        excl = jnp.where(IDX == k1, -3.0, masked)
        m2, k2 = pick(excl)
        return m1, k1, m2, jnp.minimum(k2, _N - 1)

    def gather_rows(r):
        return jnp.concatenate(
            [scr[pl.ds(i * _ROWS + r, 1), :] for i in range(5)], axis=0)

    def vec_iou(g):
        gx1, gy1, gx2, gy2, ga = g
        xx1 = jnp.maximum(gx1, X1)
        yy1 = jnp.maximum(gy1, Y1)
        xx2 = jnp.minimum(gx2, X2)
        yy2 = jnp.minimum(gy2, Y2)
        inter = jnp.clip(xx2 - xx1, 0.0, None) * jnp.clip(yy2 - yy1, 0.0, None)
        return inter / (ga + AREA - inter + 1e-9)

    masked0 = jnp.where(S > 0.0, S, -1.0)
    c0 = pick2(masked0)

    def cond(c):
        return c[1] > 0.0

    def body(c):
        masked, m1, k1, m2, k2 = c
        r1 = jax.lax.shift_right_logical(k1, 7)
        c1 = jax.lax.bitwise_and(k1, 127)
        r2 = jax.lax.shift_right_logical(k2, 7)
        c2 = jax.lax.bitwise_and(k2, 127)
        rows1 = gather_rows(r1)
        rows2 = gather_rows(r2)
        rolled1 = pltpu.roll(rows1, (128 - c1) & 127, 1)
        ga1 = [rolled1[i, 0] for i in range(5)]
        rolled2 = pltpu.roll(rows2, (128 - c2) & 127, 1)
        ga2 = [rolled2[i, 0] for i in range(5)]
        bx1, by1, bx2, by2, bar = [rows2[i:i + 1] for i in range(5)]
        rxx1 = jnp.maximum(ga1[0], bx1)
        ryy1 = jnp.maximum(ga1[1], by1)
        rxx2 = jnp.minimum(ga1[2], bx2)
        ryy2 = jnp.minimum(ga1[3], by2)
        rint = jnp.clip(rxx2 - rxx1, 0.0, None) * jnp.clip(ryy2 - ryy1, 0.0, None)
        riou = rint / (ga1[4] + bar - rint + 1e-9)
        iou_at_k2 = pltpu.roll(riou, (128 - c2) & 127, 1)[0, 0]
        valid2 = jnp.logical_and(m2 > 0.0,
                                 jnp.logical_not(iou_at_k2 > _NMS_THR))
        iou1 = vec_iou(ga1)
        iou2 = vec_iou(ga2)
        supp = (iou1 > _NMS_THR) | jnp.logical_and(iou2 > _NMS_THR, valid2)
        keepm = (IDX == k1) | jnp.logical_and(IDX == k2, valid2)
        masked = jnp.where(keepm, -2.0, jnp.where(supp, -1.0, masked))
        n1, nk1, n2, nk2 = pick2(masked)
        return masked, n1, nk1, n2, nk2

    masked, _, _, _, _ = jax.lax.while_loop(
        cond, body, (masked0,) + c0)
    objf_ref[:] = jnp.where(masked == -2.0, S, 0.0)


def _run(x2d, im_info, interpret=False):
    return pl.pallas_call(
        _yolo_body,
        out_shape=[
            jax.ShapeDtypeStruct((4, _ROWS, 128), jnp.float32),
            jax.ShapeDtypeStruct((_A, _C, _SROWS, 128), jnp.float32),
            jax.ShapeDtypeStruct((_ROWS, 128), jnp.float32),
        ],
        in_specs=[
            pl.BlockSpec(memory_space=pltpu.VMEM),
            pl.BlockSpec(memory_space=pltpu.SMEM),
        ],
        scratch_shapes=[pltpu.VMEM((5 * _ROWS, 128), jnp.float32)],
        interpret=interpret,
    )(x2d, im_info)


def kernel(x, im_info):
    x2d = x.reshape(125, _SROWS, 128)
    box4, tp, objf = _run(x2d, im_info)
    flat_boxes = box4.reshape(4, _N).T[None]
    tpf = tp.reshape(_A, _C, _HW).transpose(0, 2, 1).reshape(1, _N, _C)
    prob = jnp.concatenate([tpf, objf.reshape(1, _N, 1)], axis=-1)
    return prob, flat_boxes


# speedup vs baseline: 27.3026x; 1.0299x over previous
import jax
import jax.numpy as jnp
from jax.experimental import pallas as pl
from jax.experimental.pallas import tpu as pltpu

_A, _C, _H, _W = 5, 20, 64, 64
_HW = _H * _W
_N = _A * _HW
_ROWS = _N // 128
_SROWS = _HW // 128
_NMS_THR = 0.45
_PRE_THR = 0.005
_BIASES = ((1.08, 1.19), (3.42, 4.41), (6.63, 11.38), (9.42, 5.11), (16.62, 10.52))


def _yolo_body(x_ref, im_ref, box_ref, tp_ref, objf_ref, scr):
    im_h = im_ref[0, 0]
    im_w = im_ref[0, 1]
    r_i = jax.lax.broadcasted_iota(jnp.int32, (_SROWS, 128), 0)
    c_i = jax.lax.broadcasted_iota(jnp.int32, (_SROWS, 128), 1)
    sidx = r_i * 128 + c_i
    gxf = (sidx % _W).astype(jnp.float32)
    gyf = (sidx // _W).astype(jnp.float32)

    x1s, y1s, x2s, y2s, ss = [], [], [], [], []
    for a in range(_A):
        tx = x_ref[2 * a]
        ty = x_ref[2 * a + 1]
        tw = x_ref[2 * _A + 2 * a]
        th = x_ref[2 * _A + 2 * a + 1]
        to = x_ref[4 * _A + a]
        obj = jax.nn.sigmoid(to)
        cx = (jax.nn.sigmoid(tx) + gxf) / _W
        cy = (jax.nn.sigmoid(ty) + gyf) / _H
        bw = jnp.exp(tw) * jnp.float32(_BIASES[a][0]) / _W
        bh = jnp.exp(th) * jnp.float32(_BIASES[a][1]) / _H
        x1s.append(jnp.clip((cx - bw * 0.5) * im_w, 0.0, im_w - 1.0))
        y1s.append(jnp.clip((cy - bh * 0.5) * im_h, 0.0, im_h - 1.0))
        x2s.append(jnp.clip((cx + bw * 0.5) * im_w, 0.0, im_w - 1.0))
        y2s.append(jnp.clip((cy + bh * 0.5) * im_h, 0.0, im_h - 1.0))
        ss.append(jnp.where(obj > _PRE_THR, obj, 0.0))
        conf = x_ref[5 * _A + _C * a: 5 * _A + _C * (a + 1)]
        mx = jnp.max(conf, axis=0, keepdims=True)
        e = jnp.exp(conf - mx)
        cp = e / jnp.sum(e, axis=0, keepdims=True)
        tp_ref[a] = cp * obj[None]

    X1 = jnp.concatenate(x1s, 0)
    Y1 = jnp.concatenate(y1s, 0)
    X2 = jnp.concatenate(x2s, 0)
    Y2 = jnp.concatenate(y2s, 0)
    S = jnp.concatenate(ss, 0)
    box_ref[0] = X1
    box_ref[1] = Y1
    box_ref[2] = X2
    box_ref[3] = Y2

    AREA = jnp.clip(X2 - X1, 0.0, None) * jnp.clip(Y2 - Y1, 0.0, None)
    IDX = (jax.lax.broadcasted_iota(jnp.int32, (_ROWS, 128), 0) * 128
           + jax.lax.broadcasted_iota(jnp.int32, (_ROWS, 128), 1))
    scr[0 * _ROWS:1 * _ROWS] = X1
    scr[1 * _ROWS:2 * _ROWS] = Y1
    scr[2 * _ROWS:3 * _ROWS] = X2
    scr[3 * _ROWS:4 * _ROWS] = Y2
    scr[4 * _ROWS:5 * _ROWS] = AREA
    scr[5 * _ROWS:6 * _ROWS] = S

    def mkidx():
        return (jax.lax.broadcasted_iota(jnp.int32, (_ROWS, 128), 0) * 128
                + jax.lax.broadcasted_iota(jnp.int32, (_ROWS, 128), 1))

    def pick(masked, idx):
        m = jnp.max(masked)
        k = jnp.min(jnp.where(masked == m, idx, _N))
        return m, k

    def pick4(masked, idx):
        m1, k1 = pick(masked, idx)
        excl = jnp.where(idx == k1, -3.0, masked)
        m2, k2 = pick(excl, idx)
        excl = jnp.where(idx == k2, -3.0, excl)
        m3, k3 = pick(excl, idx)
        excl = jnp.where(idx == k3, -3.0, excl)
        m4, k4 = pick(excl, idx)
        return (m1, k1, m2, jnp.minimum(k2, _N - 1),
                m3, jnp.minimum(k3, _N - 1), m4, jnp.minimum(k4, _N - 1))

    def gather_rows(r):
        return jnp.concatenate(
            [scr[pl.ds(i * _ROWS + r, 1), :] for i in range(5)], axis=0)

    def vec_iou(g, arrs):
        gx1, gy1, gx2, gy2, ga = g
        vx1, vy1, vx2, vy2, var = arrs
        xx1 = jnp.maximum(gx1, vx1)
        yy1 = jnp.maximum(gy1, vy1)
        xx2 = jnp.minimum(gx2, vx2)
        yy2 = jnp.minimum(gy2, vy2)
        inter = jnp.clip(xx2 - xx1, 0.0, None) * jnp.clip(yy2 - yy1, 0.0, None)
        return inter / (ga + var - inter + 1e-9)

    masked0 = jnp.where(S > 0.0, S, -1.0)
    c0 = pick4(masked0, IDX)

    def cond(c):
        return c[1] > 0.0

    def mini_iou(g, rows, c):
        bx1, by1, bx2, by2, bar = [rows[i:i + 1] for i in range(5)]
        rxx1 = jnp.maximum(g[0], bx1)
        ryy1 = jnp.maximum(g[1], by1)
        rxx2 = jnp.minimum(g[2], bx2)
        ryy2 = jnp.minimum(g[3], by2)
        rint = jnp.clip(rxx2 - rxx1, 0.0, None) * jnp.clip(ryy2 - ryy1, 0.0, None)
        riou = rint / (g[4] + bar - rint + 1e-9)
        return pltpu.roll(riou, (128 - c) & 127, 1)[0, 0] > _NMS_THR

    def body(c):
        masked, m1, k1, m2, k2, m3, k3, m4, k4 = c
        ks = [k1, k2, k3, k4]
        rs = [jax.lax.shift_right_logical(k, 7) for k in ks]
        cs = [jax.lax.bitwise_and(k, 127) for k in ks]
        rows = [gather_rows(r) for r in rs]
        g = []
        for t in range(4):
            rolled = pltpu.roll(rows[t], (128 - cs[t]) & 127, 1)
            g.append([rolled[i, 0] for i in range(5)])
        s12 = mini_iou(g[0], rows[1], cs[1])
        s13 = mini_iou(g[0], rows[2], cs[2])
        s14 = mini_iou(g[0], rows[3], cs[3])
        s23 = mini_iou(g[1], rows[2], cs[2])
        s24 = mini_iou(g[1], rows[3], cs[3])
        s34 = mini_iou(g[2], rows[3], cs[3])
        nt = jnp.logical_not
        v2 = (m2 > 0.0) & nt(s12)
        v3 = (m3 > 0.0) & nt(s13) & nt(v2 & s23)
        v4 = (m4 > 0.0) & nt(s14) & nt(v2 & s24) & nt(v3 & s34)
        arrs = [scr[i * _ROWS:(i + 1) * _ROWS] for i in range(5)]
        iou1 = vec_iou(g[0], arrs)
        iou2 = vec_iou(g[1], arrs)
        iou3 = vec_iou(g[2], arrs)
        iou4 = vec_iou(g[3], arrs)
        idx = mkidx()
        supp = ((iou1 > _NMS_THR)
                | ((iou2 > _NMS_THR) & v2)
                | ((iou3 > _NMS_THR) & v3)
                | ((iou4 > _NMS_THR) & v4))
        keepm = ((idx == k1) | ((idx == k2) & v2)
                 | ((idx == k3) & v3) | ((idx == k4) & v4))
        masked = jnp.where(keepm, -2.0, jnp.where(supp, -1.0, masked))
        return (masked,) + pick4(masked, idx)

    masked = jax.lax.while_loop(cond, body, (masked0,) + c0)[0]
    objf_ref[:] = jnp.where(masked == -2.0, scr[5 * _ROWS:6 * _ROWS], 0.0)


def _run(x2d, im_info, interpret=False):
    return pl.pallas_call(
        _yolo_body,
        out_shape=[
            jax.ShapeDtypeStruct((4, _ROWS, 128), jnp.float32),
            jax.ShapeDtypeStruct((_A, _C, _SROWS, 128), jnp.float32),
            jax.ShapeDtypeStruct((_ROWS, 128), jnp.float32),
        ],
        in_specs=[
            pl.BlockSpec(memory_space=pltpu.VMEM),
            pl.BlockSpec(memory_space=pltpu.SMEM),
        ],
        scratch_shapes=[pltpu.VMEM((6 * _ROWS, 128), jnp.float32)],
        interpret=interpret,
    )(x2d, im_info)


def kernel(x, im_info):
    x2d = x.reshape(125, _SROWS, 128)
    box4, tp, objf = _run(x2d, im_info)
    flat_boxes = box4.reshape(4, _N).T[None]
    tpf = tp.reshape(_A, _C, _HW).transpose(0, 2, 1).reshape(1, _N, _C)
    prob = jnp.concatenate([tpf, objf.reshape(1, _N, 1)], axis=-1)
    return prob, flat_boxes


# speedup vs baseline: 28.9106x; 1.0589x over previous
import jax
import jax.numpy as jnp
from jax.experimental import pallas as pl
from jax.experimental.pallas import tpu as pltpu

_A, _C, _H, _W = 5, 20, 64, 64
_HW = _H * _W
_N = _A * _HW
_ROWS = _N // 128
_SROWS = _HW // 128
_NMS_THR = 0.45
_PRE_THR = 0.005
_BIASES = ((1.08, 1.19), (3.42, 4.41), (6.63, 11.38), (9.42, 5.11), (16.62, 10.52))


def _yolo_body(x_ref, im_ref, box_ref, tp_ref, objf_ref, scr, msk):
    im_h = im_ref[0, 0]
    im_w = im_ref[0, 1]
    r_i = jax.lax.broadcasted_iota(jnp.int32, (_SROWS, 128), 0)
    c_i = jax.lax.broadcasted_iota(jnp.int32, (_SROWS, 128), 1)
    sidx = r_i * 128 + c_i
    gxf = (sidx % _W).astype(jnp.float32)
    gyf = (sidx // _W).astype(jnp.float32)

    x1s, y1s, x2s, y2s, ss = [], [], [], [], []
    for a in range(_A):
        tx = x_ref[2 * a]
        ty = x_ref[2 * a + 1]
        tw = x_ref[2 * _A + 2 * a]
        th = x_ref[2 * _A + 2 * a + 1]
        to = x_ref[4 * _A + a]
        obj = jax.nn.sigmoid(to)
        cx = (jax.nn.sigmoid(tx) + gxf) / _W
        cy = (jax.nn.sigmoid(ty) + gyf) / _H
        bw = jnp.exp(tw) * jnp.float32(_BIASES[a][0]) / _W
        bh = jnp.exp(th) * jnp.float32(_BIASES[a][1]) / _H
        x1s.append(jnp.clip((cx - bw * 0.5) * im_w, 0.0, im_w - 1.0))
        y1s.append(jnp.clip((cy - bh * 0.5) * im_h, 0.0, im_h - 1.0))
        x2s.append(jnp.clip((cx + bw * 0.5) * im_w, 0.0, im_w - 1.0))
        y2s.append(jnp.clip((cy + bh * 0.5) * im_h, 0.0, im_h - 1.0))
        ss.append(jnp.where(obj > _PRE_THR, obj, 0.0))
        conf = x_ref[5 * _A + _C * a: 5 * _A + _C * (a + 1)]
        mx = jnp.max(conf, axis=0, keepdims=True)
        e = jnp.exp(conf - mx)
        cp = e / jnp.sum(e, axis=0, keepdims=True)
        tp_ref[a] = cp * obj[None]

    X1 = jnp.concatenate(x1s, 0)
    Y1 = jnp.concatenate(y1s, 0)
    X2 = jnp.concatenate(x2s, 0)
    Y2 = jnp.concatenate(y2s, 0)
    S = jnp.concatenate(ss, 0)
    box_ref[0] = X1
    box_ref[1] = Y1
    box_ref[2] = X2
    box_ref[3] = Y2

    AREA = jnp.clip(X2 - X1, 0.0, None) * jnp.clip(Y2 - Y1, 0.0, None)
    IDX = (jax.lax.broadcasted_iota(jnp.int32, (_ROWS, 128), 0) * 128
           + jax.lax.broadcasted_iota(jnp.int32, (_ROWS, 128), 1))
    scr[0 * _ROWS:1 * _ROWS] = X1
    scr[1 * _ROWS:2 * _ROWS] = Y1
    scr[2 * _ROWS:3 * _ROWS] = X2
    scr[3 * _ROWS:4 * _ROWS] = Y2
    scr[4 * _ROWS:5 * _ROWS] = AREA
    scr[5 * _ROWS:6 * _ROWS] = S

    def mkidx():
        return (jax.lax.broadcasted_iota(jnp.int32, (_ROWS, 128), 0) * 128
                + jax.lax.broadcasted_iota(jnp.int32, (_ROWS, 128), 1))

    _LANE = jax.lax.broadcasted_iota(jnp.int32, (1, 128), 1)

    def fixup(k, cond_scalar, val):
        r = jax.lax.shift_right_logical(k, 7)
        c = jax.lax.bitwise_and(k, 127)
        row = msk[pl.ds(r, 1), :]
        msk[pl.ds(r, 1), :] = jnp.where(
            jnp.logical_and(_LANE == c, cond_scalar), val, row)

    def pick(masked, idx):
        m = jnp.max(masked)
        k = jnp.min(jnp.where(masked == m, idx, _N))
        return m, k

    def pick4(idx):
        m1, k1 = pick(msk[:], idx)
        fixup(k1, m1 > 0.0, -3.0)
        m2, k2 = pick(msk[:], idx)
        k2 = jnp.minimum(k2, _N - 1)
        fixup(k2, m2 > 0.0, -3.0)
        m3, k3 = pick(msk[:], idx)
        k3 = jnp.minimum(k3, _N - 1)
        fixup(k3, m3 > 0.0, -3.0)
        m4, k4 = pick(msk[:], idx)
        return (m1, k1, m2, k2, m3, k3, m4, jnp.minimum(k4, _N - 1))

    def gather_rows(r):
        return jnp.concatenate(
            [scr[pl.ds(i * _ROWS + r, 1), :] for i in range(5)], axis=0)

    def vec_iou(g, arrs):
        gx1, gy1, gx2, gy2, ga = g
        vx1, vy1, vx2, vy2, var = arrs
        xx1 = jnp.maximum(gx1, vx1)
        yy1 = jnp.maximum(gy1, vy1)
        xx2 = jnp.minimum(gx2, vx2)
        yy2 = jnp.minimum(gy2, vy2)
        inter = jnp.clip(xx2 - xx1, 0.0, None) * jnp.clip(yy2 - yy1, 0.0, None)
        return inter / (ga + var - inter + 1e-9)

    msk[:] = jnp.where(S > 0.0, S, -1.0)
    c0 = pick4(IDX)

    def cond(c):
        return c[0] > 0.0

    def mini_iou(g, rows, c):
        bx1, by1, bx2, by2, bar = [rows[i:i + 1] for i in range(5)]
        rxx1 = jnp.maximum(g[0], bx1)
        ryy1 = jnp.maximum(g[1], by1)
        rxx2 = jnp.minimum(g[2], bx2)
        ryy2 = jnp.minimum(g[3], by2)
        rint = jnp.clip(rxx2 - rxx1, 0.0, None) * jnp.clip(ryy2 - ryy1, 0.0, None)
        riou = rint / (g[4] + bar - rint + 1e-9)
        return pltpu.roll(riou, (128 - c) & 127, 1)[0, 0] > _NMS_THR

    def body(c):
        m1, k1, m2, k2, m3, k3, m4, k4 = c
        ks = [k1, k2, k3, k4]
        rs = [jax.lax.shift_right_logical(k, 7) for k in ks]
        cs = [jax.lax.bitwise_and(k, 127) for k in ks]
        rows = [gather_rows(r) for r in rs]
        g = []
        for t in range(4):
            rolled = pltpu.roll(rows[t], (128 - cs[t]) & 127, 1)
            g.append([rolled[i, 0] for i in range(5)])
        s12 = mini_iou(g[0], rows[1], cs[1])
        s13 = mini_iou(g[0], rows[2], cs[2])
        s14 = mini_iou(g[0], rows[3], cs[3])
        s23 = mini_iou(g[1], rows[2], cs[2])
        s24 = mini_iou(g[1], rows[3], cs[3])
        s34 = mini_iou(g[2], rows[3], cs[3])
        nt = jnp.logical_not
        v2 = (m2 > 0.0) & nt(s12)
        v3 = (m3 > 0.0) & nt(s13) & nt(v2 & s23)
        v4 = (m4 > 0.0) & nt(s14) & nt(v2 & s24) & nt(v3 & s34)
        v2f = jnp.where(v2, 1.0, 0.0)
        v3f = jnp.where(v3, 1.0, 0.0)
        v4f = jnp.where(v4, 1.0, 0.0)
        arrs = [scr[i * _ROWS:(i + 1) * _ROWS] for i in range(5)]
        iou1 = vec_iou(g[0], arrs)
        iou2 = vec_iou(g[1], arrs)
        iou3 = vec_iou(g[2], arrs)
        iou4 = vec_iou(g[3], arrs)
        mx = jnp.maximum(jnp.maximum(iou1, iou2 * v2f),
                         jnp.maximum(iou3 * v3f, iou4 * v4f))
        msk[:] = jnp.where(mx > _NMS_THR, -1.0, msk[:])
        fixup(k1, m1 > 0.0, -2.0)
        fixup(k2, v2, -2.0)
        fixup(k3, v3, -2.0)
        fixup(k4, v4, -2.0)
        return pick4(mkidx())

    jax.lax.while_loop(cond, body, c0)
    objf_ref[:] = jnp.where(msk[:] == -2.0, scr[5 * _ROWS:6 * _ROWS], 0.0)


def _run(x2d, im_info, interpret=False):
    return pl.pallas_call(
        _yolo_body,
        out_shape=[
            jax.ShapeDtypeStruct((4, _ROWS, 128), jnp.float32),
            jax.ShapeDtypeStruct((_A, _C, _SROWS, 128), jnp.float32),
            jax.ShapeDtypeStruct((_ROWS, 128), jnp.float32),
        ],
        in_specs=[
            pl.BlockSpec(memory_space=pltpu.VMEM),
            pl.BlockSpec(memory_space=pltpu.SMEM),
        ],
        scratch_shapes=[pltpu.VMEM((6 * _ROWS, 128), jnp.float32),
                        pltpu.VMEM((_ROWS, 128), jnp.float32)],
        interpret=interpret,
    )(x2d, im_info)


def kernel(x, im_info):
    x2d = x.reshape(125, _SROWS, 128)
    box4, tp, objf = _run(x2d, im_info)
    flat_boxes = box4.reshape(4, _N).T[None]
    tpf = tp.reshape(_A, _C, _HW).transpose(0, 2, 1).reshape(1, _N, _C)
    prob = jnp.concatenate([tpf, objf.reshape(1, _N, 1)], axis=-1)
    return prob, flat_boxes


# speedup vs baseline: 28.9467x; 1.0012x over previous
import jax
import jax.numpy as jnp
from jax.experimental import pallas as pl
from jax.experimental.pallas import tpu as pltpu

_A, _C, _H, _W = 5, 20, 64, 64
_HW = _H * _W
_N = _A * _HW
_ROWS = _N // 128
_SROWS = _HW // 128
_NMS_THR = 0.45
_PRE_THR = 0.005
_BIASES = ((1.08, 1.19), (3.42, 4.41), (6.63, 11.38), (9.42, 5.11), (16.62, 10.52))


def _yolo_body(x_ref, im_ref, box_ref, tp_ref, objf_ref, scr, msk):
    im_h = im_ref[0, 0]
    im_w = im_ref[0, 1]
    r_i = jax.lax.broadcasted_iota(jnp.int32, (_SROWS, 128), 0)
    c_i = jax.lax.broadcasted_iota(jnp.int32, (_SROWS, 128), 1)
    sidx = r_i * 128 + c_i
    gxf = (sidx % _W).astype(jnp.float32)
    gyf = (sidx // _W).astype(jnp.float32)

    x1s, y1s, x2s, y2s, ss = [], [], [], [], []
    for a in range(_A):
        tx = x_ref[2 * a]
        ty = x_ref[2 * a + 1]
        tw = x_ref[2 * _A + 2 * a]
        th = x_ref[2 * _A + 2 * a + 1]
        to = x_ref[4 * _A + a]
        obj = jax.nn.sigmoid(to)
        cx = (jax.nn.sigmoid(tx) + gxf) / _W
        cy = (jax.nn.sigmoid(ty) + gyf) / _H
        bw = jnp.exp(tw) * jnp.float32(_BIASES[a][0]) / _W
        bh = jnp.exp(th) * jnp.float32(_BIASES[a][1]) / _H
        x1s.append(jnp.clip((cx - bw * 0.5) * im_w, 0.0, im_w - 1.0))
        y1s.append(jnp.clip((cy - bh * 0.5) * im_h, 0.0, im_h - 1.0))
        x2s.append(jnp.clip((cx + bw * 0.5) * im_w, 0.0, im_w - 1.0))
        y2s.append(jnp.clip((cy + bh * 0.5) * im_h, 0.0, im_h - 1.0))
        ss.append(jnp.where(obj > _PRE_THR, obj, 0.0))
        conf = x_ref[5 * _A + _C * a: 5 * _A + _C * (a + 1)]
        mx = jnp.max(conf, axis=0, keepdims=True)
        e = jnp.exp(conf - mx)
        cp = e / jnp.sum(e, axis=0, keepdims=True)
        tp_ref[a] = cp * obj[None]

    X1 = jnp.concatenate(x1s, 0)
    Y1 = jnp.concatenate(y1s, 0)
    X2 = jnp.concatenate(x2s, 0)
    Y2 = jnp.concatenate(y2s, 0)
    S = jnp.concatenate(ss, 0)
    box_ref[0] = X1
    box_ref[1] = Y1
    box_ref[2] = X2
    box_ref[3] = Y2

    AREA = jnp.clip(X2 - X1, 0.0, None) * jnp.clip(Y2 - Y1, 0.0, None)
    IDX = (jax.lax.broadcasted_iota(jnp.int32, (_ROWS, 128), 0) * 128
           + jax.lax.broadcasted_iota(jnp.int32, (_ROWS, 128), 1))
    scr[0 * _ROWS:1 * _ROWS] = X1
    scr[1 * _ROWS:2 * _ROWS] = Y1
    scr[2 * _ROWS:3 * _ROWS] = X2
    scr[3 * _ROWS:4 * _ROWS] = Y2
    scr[4 * _ROWS:5 * _ROWS] = AREA
    scr[5 * _ROWS:6 * _ROWS] = S

    def mkidx():
        return (jax.lax.broadcasted_iota(jnp.int32, (_ROWS, 128), 0) * 128
                + jax.lax.broadcasted_iota(jnp.int32, (_ROWS, 128), 1))

    _LANE = jax.lax.broadcasted_iota(jnp.int32, (1, 128), 1)

    def fixup(k, cond_scalar, val):
        r = jax.lax.shift_right_logical(k, 7)
        c = jax.lax.bitwise_and(k, 127)
        row = msk[pl.ds(r, 1), :]
        msk[pl.ds(r, 1), :] = jnp.where(
            jnp.logical_and(_LANE == c, cond_scalar), val, row)

    def pick(masked, idx):
        m = jnp.max(masked)
        k = jnp.min(jnp.where(masked == m, idx, _N))
        return m, k

    def pick4(idx):
        m1, k1 = pick(msk[:], idx)
        fixup(k1, m1 > 0.0, -3.0)
        m2, k2 = pick(msk[:], idx)
        k2 = jnp.minimum(k2, _N - 1)
        fixup(k2, m2 > 0.0, -3.0)
        m3, k3 = pick(msk[:], idx)
        k3 = jnp.minimum(k3, _N - 1)
        fixup(k3, m3 > 0.0, -3.0)
        m4, k4 = pick(msk[:], idx)
        return (m1, k1, m2, k2, m3, k3, m4, jnp.minimum(k4, _N - 1))

    def gather_rows(r):
        return jnp.concatenate(
            [scr[pl.ds(i * _ROWS + r, 1), :] for i in range(5)], axis=0)

    def vec_iou(g, arrs):
        gx1, gy1, gx2, gy2, ga = g
        vx1, vy1, vx2, vy2, var = arrs
        xx1 = jnp.maximum(gx1, vx1)
        yy1 = jnp.maximum(gy1, vy1)
        xx2 = jnp.minimum(gx2, vx2)
        yy2 = jnp.minimum(gy2, vy2)
        inter = jnp.clip(xx2 - xx1, 0.0, None) * jnp.clip(yy2 - yy1, 0.0, None)
        return inter / (ga + var - inter + 1e-9)

    msk[:] = jnp.where(S > 0.0, S, -1.0)
    c0 = pick4(IDX)

    def cond(c):
        return c[0] > 0.0

    def mini_iou(g, rows, c):
        bx1, by1, bx2, by2, bar = [rows[i:i + 1] for i in range(5)]
        rxx1 = jnp.maximum(g[0], bx1)
        ryy1 = jnp.maximum(g[1], by1)
        rxx2 = jnp.minimum(g[2], bx2)
        ryy2 = jnp.minimum(g[3], by2)
        rint = jnp.clip(rxx2 - rxx1, 0.0, None) * jnp.clip(ryy2 - ryy1, 0.0, None)
        riou = rint / (g[4] + bar - rint + 1e-9)
        return pltpu.roll(riou, (128 - c) & 127, 1)[0, 0] > _NMS_THR

    def body(c):
        m1, k1, m2, k2, m3, k3, m4, k4 = c
        ks = [k1, k2, k3, k4]
        rs = [jax.lax.shift_right_logical(k, 7) for k in ks]
        cs = [jax.lax.bitwise_and(k, 127) for k in ks]
        rows = [gather_rows(r) for r in rs]
        g = []
        for t in range(4):
            rolled = pltpu.roll(rows[t], (128 - cs[t]) & 127, 1)
            g.append([rolled[i, 0] for i in range(5)])
        s12 = mini_iou(g[0], rows[1], cs[1])
        s13 = mini_iou(g[0], rows[2], cs[2])
        s14 = mini_iou(g[0], rows[3], cs[3])
        s23 = mini_iou(g[1], rows[2], cs[2])
        s24 = mini_iou(g[1], rows[3], cs[3])
        s34 = mini_iou(g[2], rows[3], cs[3])
        nt = jnp.logical_not
        v2 = (m2 > 0.0) & nt(s12)
        v3 = (m3 > 0.0) & nt(s13) & nt(v2 & s23)
        v4 = (m4 > 0.0) & nt(s14) & nt(v2 & s24) & nt(v3 & s34)
        v2f = jnp.where(v2, 1.0, 0.0)
        v3f = jnp.where(v3, 1.0, 0.0)
        v4f = jnp.where(v4, 1.0, 0.0)
        arrs = [scr[i * _ROWS:(i + 1) * _ROWS] for i in range(5)]
        mx = vec_iou(g[0], arrs)
        mx = jnp.maximum(mx, vec_iou(g[1], arrs) * v2f)
        mx = jnp.maximum(mx, vec_iou(g[2], arrs) * v3f)
        mx = jnp.maximum(mx, vec_iou(g[3], arrs) * v4f)
        msk[:] = jnp.where(mx > _NMS_THR, -1.0, msk[:])
        fixup(k1, m1 > 0.0, -2.0)
        fixup(k2, v2, -2.0)
        fixup(k3, v3, -2.0)
        fixup(k4, v4, -2.0)
        return pick4(mkidx())

    jax.lax.while_loop(cond, body, c0)
    objf_ref[:] = jnp.where(msk[:] == -2.0, scr[5 * _ROWS:6 * _ROWS], 0.0)


def _run(x2d, im_info, interpret=False):
    return pl.pallas_call(
        _yolo_body,
        out_shape=[
            jax.ShapeDtypeStruct((4, _ROWS, 128), jnp.float32),
            jax.ShapeDtypeStruct((_A, _C, _SROWS, 128), jnp.float32),
            jax.ShapeDtypeStruct((_ROWS, 128), jnp.float32),
        ],
        in_specs=[
            pl.BlockSpec(memory_space=pltpu.VMEM),
            pl.BlockSpec(memory_space=pltpu.SMEM),
        ],
        scratch_shapes=[pltpu.VMEM((6 * _ROWS, 128), jnp.float32),
                        pltpu.VMEM((_ROWS, 128), jnp.float32)],
        interpret=interpret,
    )(x2d, im_info)


def kernel(x, im_info):
    x2d = x.reshape(125, _SROWS, 128)
    box4, tp, objf = _run(x2d, im_info)
    flat_boxes = box4.reshape(4, _N).T[None]
    tpf = tp.reshape(_A, _C, _HW).transpose(0, 2, 1).reshape(1, _N, _C)
    prob = jnp.concatenate([tpf, objf.reshape(1, _N, 1)], axis=-1)
    return prob, flat_boxes
